# baseline (speedup 1.0000x reference)
import jax
import jax.numpy as jnp
from jax import lax
from jax.experimental import pallas as pl
from jax.experimental.pallas import tpu as pltpu
from jax.experimental.pallas import tpu_sc as plsc

TOKENS = 4096
D_IN = 2048
D_HID = 2048
N_EXP = 8
K_TOP = 2

BM_A = 512
BN_C = 256

SC_NC = 2
SC_NS = 16
SC_L = 16
SC_NW = SC_NC * SC_NS
SC_TPW = TOKENS // SC_NW


def _gating_body(x_ref, wg_ref, bg_ref, pt_ref, xbf_ref):
    xv = x_ref[...]
    logits = jnp.dot(xv, wg_ref[...], preferred_element_type=jnp.float32)
    logits = logits + bg_ref[...]
    m = jnp.max(logits, axis=1, keepdims=True)
    e = jnp.exp(logits - m)
    p = e / jnp.sum(e, axis=1, keepdims=True)
    pt_ref[...] = p.T
    xbf_ref[...] = xv.astype(jnp.bfloat16)


def _router_body(pt_hbm, s0_hbm, s1_hbm, i0_hbm, i1_hbm, pv, sv0, sv1,
                 iv0, iv1):
    wid = lax.axis_index("s") * SC_NC + lax.axis_index("c")
    base = wid * SC_TPW
    pltpu.sync_copy(pt_hbm.at[:, pl.ds(base, SC_TPW)], pv)
    lane = lax.iota(jnp.int32, SC_L)
    neg_inf = jnp.float32(-jnp.inf)
    for j in range(SC_TPW // SC_L):
        vs = [pv[e, pl.ds(j * SC_L, SC_L)] for e in range(N_EXP)]
        m1 = vs[0]
        for e in range(1, N_EXP):
            m1 = jnp.maximum(m1, vs[e])
        fi = jnp.full((SC_L,), N_EXP, jnp.int32)
        for e in range(N_EXP):
            fi = jnp.minimum(fi, jnp.where(vs[e] == m1, e, N_EXP))
        m2 = jnp.full((SC_L,), neg_inf)
        for e in range(N_EXP):
            m2 = jnp.maximum(m2, jnp.where(fi == e, neg_inf, vs[e]))
        sv0[pl.ds(j * SC_L, SC_L)] = m1
        sv1[pl.ds(j * SC_L, SC_L)] = m2
        if j == 0:
            @pl.when(wid == 0)
            def _():
                si = jnp.full((SC_L,), N_EXP, jnp.int32)
                for e in range(N_EXP):
                    si = jnp.minimum(
                        si, jnp.where((vs[e] == m2) & (fi != e), e, N_EXP))
                iv0[...] = fi
                iv1[...] = si
                pltpu.sync_copy(iv0, i0_hbm)
                pltpu.sync_copy(iv1, i1_hbm)
    pltpu.sync_copy(sv0, s0_hbm.at[pl.ds(base, SC_TPW)])
    pltpu.sync_copy(sv1, s1_hbm.at[pl.ds(base, SC_TPW)])


def _expert_body(i0_ref, i1_ref, x_ref, w0_ref, w1_ref, b0_ref, b1_ref,
                 s0_ref, s1_ref, o_ref):
    xb = x_ref[...]
    d0 = jnp.dot(xb, w0_ref[0].astype(jnp.bfloat16),
                 preferred_element_type=jnp.float32)
    d1 = jnp.dot(xb, w1_ref[0].astype(jnp.bfloat16),
                 preferred_element_type=jnp.float32)
    s0 = s0_ref[...]
    s1 = s1_ref[...]
    o_ref[...] = s0 * (d0 + b0_ref[0]) + s1 * (d1 + b1_ref[0])


def kernel(x, Wg, bg, W, b):
    bg2 = bg.reshape(1, N_EXP)
    b3 = b.reshape(N_EXP, 1, D_HID)

    n_blk = TOKENS // BM_A
    probs_t, xbf = pl.pallas_call(
        _gating_body,
        grid=(n_blk,),
        in_specs=[
            pl.BlockSpec((BM_A, D_IN), lambda i: (i, 0)),
            pl.BlockSpec((D_IN, N_EXP), lambda i: (0, 0)),
            pl.BlockSpec((1, N_EXP), lambda i: (0, 0)),
        ],
        out_specs=[
            pl.BlockSpec((N_EXP, BM_A), lambda i: (0, i)),
            pl.BlockSpec((BM_A, D_IN), lambda i: (i, 0)),
        ],
        out_shape=[
            jax.ShapeDtypeStruct((N_EXP, TOKENS), jnp.float32),
            jax.ShapeDtypeStruct((TOKENS, D_IN), jnp.bfloat16),
        ],
        compiler_params=pltpu.CompilerParams(
            dimension_semantics=("arbitrary",)),
    )(x, Wg, bg2)

    s0, s1, i0, i1 = pl.kernel(
        _router_body,
        out_type=[
            jax.ShapeDtypeStruct((TOKENS,), jnp.float32),
            jax.ShapeDtypeStruct((TOKENS,), jnp.float32),
            jax.ShapeDtypeStruct((SC_L,), jnp.int32),
            jax.ShapeDtypeStruct((SC_L,), jnp.int32),
        ],
        mesh=plsc.VectorSubcoreMesh(core_axis_name="c", subcore_axis_name="s"),
        scratch_types=[
            pltpu.VMEM((N_EXP, SC_TPW), jnp.float32),
            pltpu.VMEM((SC_TPW,), jnp.float32),
            pltpu.VMEM((SC_TPW,), jnp.float32),
            pltpu.VMEM((SC_L,), jnp.int32),
            pltpu.VMEM((SC_L,), jnp.int32),
        ],
    )(probs_t)
    s0c = s0.reshape(TOKENS, 1)
    s1c = s1.reshape(TOKENS, 1)

    out = pl.pallas_call(
        _expert_body,
        grid_spec=pltpu.PrefetchScalarGridSpec(
            num_scalar_prefetch=2,
            grid=(D_HID // BN_C,),
            in_specs=[
                pl.BlockSpec((TOKENS, D_IN), lambda n, i0, i1: (0, 0)),
                pl.BlockSpec((1, D_IN, BN_C), lambda n, i0, i1: (i0[0], 0, n)),
                pl.BlockSpec((1, D_IN, BN_C), lambda n, i0, i1: (i1[0], 0, n)),
                pl.BlockSpec((1, 1, BN_C), lambda n, i0, i1: (i0[0], 0, n)),
                pl.BlockSpec((1, 1, BN_C), lambda n, i0, i1: (i1[0], 0, n)),
                pl.BlockSpec((TOKENS, 1), lambda n, i0, i1: (0, 0)),
                pl.BlockSpec((TOKENS, 1), lambda n, i0, i1: (0, 0)),
            ],
            out_specs=pl.BlockSpec((TOKENS, BN_C), lambda n, i0, i1: (0, n)),
        ),
        out_shape=jax.ShapeDtypeStruct((TOKENS, D_HID), jnp.float32),
        compiler_params=pltpu.CompilerParams(
            dimension_semantics=("arbitrary",)),
    )(i0, i1, xbf, W, W, b3, b3, s0c, s1c)
    return out


# speedup vs baseline: 1.0267x; 1.0267x over previous
import jax
import jax.numpy as jnp
from jax import lax
from jax.experimental import pallas as pl
from jax.experimental.pallas import tpu as pltpu
from jax.experimental.pallas import tpu_sc as plsc

TOKENS = 4096
D_IN = 2048
D_HID = 2048
N_EXP = 8
K_TOP = 2

BM_A = 512
BN_C = 256

SC_NC = 2
SC_NS = 16
SC_L = 16
SC_NW = SC_NC * SC_NS
SC_TPW = TOKENS // SC_NW


def _gating_body(x_ref, wg_ref, bg_ref, pt_ref, xbf_ref):
    xv = x_ref[...]
    logits = jnp.dot(xv, wg_ref[...], preferred_element_type=jnp.float32)
    logits = logits + bg_ref[...]
    m = jnp.max(logits, axis=1, keepdims=True)
    e = jnp.exp(logits - m)
    p = e / jnp.sum(e, axis=1, keepdims=True)
    pt_ref[...] = p.T
    xbf_ref[...] = xv.astype(jnp.bfloat16)


def _router_body(pt_hbm, s0_hbm, s1_hbm, i0_hbm, i1_hbm, pv, sv0, sv1,
                 iv0, iv1):
    wid = lax.axis_index("s") * SC_NC + lax.axis_index("c")
    base = wid * SC_TPW
    pltpu.sync_copy(pt_hbm.at[:, pl.ds(base, SC_TPW)], pv)
    lane = lax.iota(jnp.int32, SC_L)
    neg_inf = jnp.float32(-jnp.inf)
    for j in range(SC_TPW // SC_L):
        vs = [pv[e, pl.ds(j * SC_L, SC_L)] for e in range(N_EXP)]
        m1 = vs[0]
        for e in range(1, N_EXP):
            m1 = jnp.maximum(m1, vs[e])
        fi = jnp.full((SC_L,), N_EXP, jnp.int32)
        for e in range(N_EXP):
            fi = jnp.minimum(fi, jnp.where(vs[e] == m1, e, N_EXP))
        m2 = jnp.full((SC_L,), neg_inf)
        for e in range(N_EXP):
            m2 = jnp.maximum(m2, jnp.where(fi == e, neg_inf, vs[e]))
        sv0[pl.ds(j * SC_L, SC_L)] = m1
        sv1[pl.ds(j * SC_L, SC_L)] = m2
        if j == 0:
            @pl.when(wid == 0)
            def _():
                si = jnp.full((SC_L,), N_EXP, jnp.int32)
                for e in range(N_EXP):
                    si = jnp.minimum(
                        si, jnp.where((vs[e] == m2) & (fi != e), e, N_EXP))
                iv0[...] = fi
                iv1[...] = si
                pltpu.sync_copy(iv0, i0_hbm)
                pltpu.sync_copy(iv1, i1_hbm)
    pltpu.sync_copy(sv0, s0_hbm.at[pl.ds(base, SC_TPW)])
    pltpu.sync_copy(sv1, s1_hbm.at[pl.ds(base, SC_TPW)])


def _expert_body(i0_ref, i1_ref, x_ref, w0_ref, w1_ref, b0_ref, b1_ref,
                 s_ref, o_ref):
    xb = x_ref[...]
    d0 = jnp.dot(xb, w0_ref[0].astype(jnp.bfloat16),
                 preferred_element_type=jnp.float32)
    d1 = jnp.dot(xb, w1_ref[0].astype(jnp.bfloat16),
                 preferred_element_type=jnp.float32)
    sv = s_ref[...]
    s0 = sv[:, 0:1]
    s1 = sv[:, 1:2]
    o_ref[...] = s0 * (d0 + b0_ref[0]) + s1 * (d1 + b1_ref[0])


def kernel(x, Wg, bg, W, b):
    bg2 = bg.reshape(1, N_EXP)
    b3 = b.reshape(N_EXP, 1, D_HID)

    n_blk = TOKENS // BM_A
    probs_t, xbf = pl.pallas_call(
        _gating_body,
        grid=(n_blk,),
        in_specs=[
            pl.BlockSpec((BM_A, D_IN), lambda i: (i, 0)),
            pl.BlockSpec((D_IN, N_EXP), lambda i: (0, 0)),
            pl.BlockSpec((1, N_EXP), lambda i: (0, 0)),
        ],
        out_specs=[
            pl.BlockSpec((N_EXP, BM_A), lambda i: (0, i)),
            pl.BlockSpec((BM_A, D_IN), lambda i: (i, 0)),
        ],
        out_shape=[
            jax.ShapeDtypeStruct((N_EXP, TOKENS), jnp.float32),
            jax.ShapeDtypeStruct((TOKENS, D_IN), jnp.bfloat16),
        ],
        compiler_params=pltpu.CompilerParams(
            dimension_semantics=("arbitrary",)),
    )(x, Wg, bg2)

    s0, s1, i0, i1 = pl.kernel(
        _router_body,
        out_type=[
            jax.ShapeDtypeStruct((TOKENS,), jnp.float32),
            jax.ShapeDtypeStruct((TOKENS,), jnp.float32),
            jax.ShapeDtypeStruct((SC_L,), jnp.int32),
            jax.ShapeDtypeStruct((SC_L,), jnp.int32),
        ],
        mesh=plsc.VectorSubcoreMesh(core_axis_name="c", subcore_axis_name="s"),
        scratch_types=[
            pltpu.VMEM((N_EXP, SC_TPW), jnp.float32),
            pltpu.VMEM((SC_TPW,), jnp.float32),
            pltpu.VMEM((SC_TPW,), jnp.float32),
            pltpu.VMEM((SC_L,), jnp.int32),
            pltpu.VMEM((SC_L,), jnp.int32),
        ],
    )(probs_t)
    s2 = jnp.stack([s0, s1], axis=-1)

    out = pl.pallas_call(
        _expert_body,
        grid_spec=pltpu.PrefetchScalarGridSpec(
            num_scalar_prefetch=2,
            grid=(D_HID // BN_C,),
            in_specs=[
                pl.BlockSpec((TOKENS, D_IN), lambda n, i0, i1: (0, 0)),
                pl.BlockSpec((1, D_IN, BN_C), lambda n, i0, i1: (i0[0], 0, n)),
                pl.BlockSpec((1, D_IN, BN_C), lambda n, i0, i1: (i1[0], 0, n)),
                pl.BlockSpec((1, 1, BN_C), lambda n, i0, i1: (i0[0], 0, n)),
                pl.BlockSpec((1, 1, BN_C), lambda n, i0, i1: (i1[0], 0, n)),
                pl.BlockSpec((TOKENS, K_TOP), lambda n, i0, i1: (0, 0)),
            ],
            out_specs=pl.BlockSpec((TOKENS, BN_C), lambda n, i0, i1: (0, n)),
        ),
        out_shape=jax.ShapeDtypeStruct((TOKENS, D_HID), jnp.float32),
        compiler_params=pltpu.CompilerParams(
            dimension_semantics=("arbitrary",)),
    )(i0, i1, xbf, W, W, b3, b3, s2)
    return out


# speedup vs baseline: 1.0670x; 1.0393x over previous
import jax
import jax.numpy as jnp
from jax import lax
from jax.experimental import pallas as pl
from jax.experimental.pallas import tpu as pltpu
from jax.experimental.pallas import tpu_sc as plsc

TOKENS = 4096
D_IN = 2048
D_HID = 2048
N_EXP = 8
K_TOP = 2

BM_A = 1024
BN_C = 256

SC_NC = 2
SC_NS = 16
SC_L = 16
SC_NW = SC_NC * SC_NS
SC_TPW = TOKENS // SC_NW


def _gating_body(x_ref, wg_ref, bg_ref, pt_ref, xbf_ref):
    xv = x_ref[...]
    logits = jnp.dot(xv, wg_ref[...], preferred_element_type=jnp.float32)
    logits = logits + bg_ref[...]
    m = jnp.max(logits, axis=1, keepdims=True)
    e = jnp.exp(logits - m)
    p = e / jnp.sum(e, axis=1, keepdims=True)
    pt_ref[...] = p.T
    xbf_ref[...] = xv.astype(jnp.bfloat16)


def _router_body(pt_hbm, s0_hbm, s1_hbm, i0_hbm, i1_hbm, pv, sv0, sv1,
                 iv0, iv1):
    wid = lax.axis_index("s") * SC_NC + lax.axis_index("c")
    base = wid * SC_TPW
    pltpu.sync_copy(pt_hbm.at[:, pl.ds(base, SC_TPW)], pv)
    lane = lax.iota(jnp.int32, SC_L)
    neg_inf = jnp.float32(-jnp.inf)
    for j in range(SC_TPW // SC_L):
        vs = [pv[e, pl.ds(j * SC_L, SC_L)] for e in range(N_EXP)]
        m1 = vs[0]
        for e in range(1, N_EXP):
            m1 = jnp.maximum(m1, vs[e])
        fi = jnp.full((SC_L,), N_EXP, jnp.int32)
        for e in range(N_EXP):
            fi = jnp.minimum(fi, jnp.where(vs[e] == m1, e, N_EXP))
        m2 = jnp.full((SC_L,), neg_inf)
        for e in range(N_EXP):
            m2 = jnp.maximum(m2, jnp.where(fi == e, neg_inf, vs[e]))
        sv0[pl.ds(j * SC_L, SC_L)] = m1
        sv1[pl.ds(j * SC_L, SC_L)] = m2
        if j == 0:
            @pl.when(wid == 0)
            def _():
                si = jnp.full((SC_L,), N_EXP, jnp.int32)
                for e in range(N_EXP):
                    si = jnp.minimum(
                        si, jnp.where((vs[e] == m2) & (fi != e), e, N_EXP))
                iv0[...] = fi
                iv1[...] = si
                pltpu.sync_copy(iv0, i0_hbm)
                pltpu.sync_copy(iv1, i1_hbm)
    pltpu.sync_copy(sv0, s0_hbm.at[pl.ds(base, SC_TPW)])
    pltpu.sync_copy(sv1, s1_hbm.at[pl.ds(base, SC_TPW)])


def _expert_body(i0_ref, i1_ref, x_ref, w0_ref, w1_ref, b0_ref, b1_ref,
                 s0_ref, s1_ref, o_ref):
    xb = x_ref[...]
    d0 = jnp.dot(xb, w0_ref[0].astype(jnp.bfloat16),
                 preferred_element_type=jnp.float32)
    d1 = jnp.dot(xb, w1_ref[0].astype(jnp.bfloat16),
                 preferred_element_type=jnp.float32)
    s0 = s0_ref[...][:, None]
    s1 = s1_ref[...][:, None]
    o_ref[...] = s0 * (d0 + b0_ref[0]) + s1 * (d1 + b1_ref[0])


def kernel(x, Wg, bg, W, b):
    bg2 = bg.reshape(1, N_EXP)
    b3 = b.reshape(N_EXP, 1, D_HID)

    n_blk = TOKENS // BM_A
    probs_t, xbf = pl.pallas_call(
        _gating_body,
        grid=(n_blk,),
        in_specs=[
            pl.BlockSpec((BM_A, D_IN), lambda i: (i, 0)),
            pl.BlockSpec((D_IN, N_EXP), lambda i: (0, 0)),
            pl.BlockSpec((1, N_EXP), lambda i: (0, 0)),
        ],
        out_specs=[
            pl.BlockSpec((N_EXP, BM_A), lambda i: (0, i)),
            pl.BlockSpec((BM_A, D_IN), lambda i: (i, 0)),
        ],
        out_shape=[
            jax.ShapeDtypeStruct((N_EXP, TOKENS), jnp.float32),
            jax.ShapeDtypeStruct((TOKENS, D_IN), jnp.bfloat16),
        ],
        compiler_params=pltpu.CompilerParams(
            dimension_semantics=("arbitrary",)),
    )(x, Wg, bg2)

    s0, s1, i0, i1 = pl.kernel(
        _router_body,
        out_type=[
            jax.ShapeDtypeStruct((TOKENS,), jnp.float32),
            jax.ShapeDtypeStruct((TOKENS,), jnp.float32),
            jax.ShapeDtypeStruct((SC_L,), jnp.int32),
            jax.ShapeDtypeStruct((SC_L,), jnp.int32),
        ],
        mesh=plsc.VectorSubcoreMesh(core_axis_name="c", subcore_axis_name="s"),
        scratch_types=[
            pltpu.VMEM((N_EXP, SC_TPW), jnp.float32),
            pltpu.VMEM((SC_TPW,), jnp.float32),
            pltpu.VMEM((SC_TPW,), jnp.float32),
            pltpu.VMEM((SC_L,), jnp.int32),
            pltpu.VMEM((SC_L,), jnp.int32),
        ],
    )(probs_t)

    out = pl.pallas_call(
        _expert_body,
        grid_spec=pltpu.PrefetchScalarGridSpec(
            num_scalar_prefetch=2,
            grid=(D_HID // BN_C,),
            in_specs=[
                pl.BlockSpec((TOKENS, D_IN), lambda n, i0, i1: (0, 0)),
                pl.BlockSpec((1, D_IN, BN_C), lambda n, i0, i1: (i0[0], 0, n)),
                pl.BlockSpec((1, D_IN, BN_C), lambda n, i0, i1: (i1[0], 0, n)),
                pl.BlockSpec((1, 1, BN_C), lambda n, i0, i1: (i0[0], 0, n)),
                pl.BlockSpec((1, 1, BN_C), lambda n, i0, i1: (i1[0], 0, n)),
                pl.BlockSpec((TOKENS,), lambda n, i0, i1: (0,)),
                pl.BlockSpec((TOKENS,), lambda n, i0, i1: (0,)),
            ],
            out_specs=pl.BlockSpec((TOKENS, BN_C), lambda n, i0, i1: (0, n)),
        ),
        out_shape=jax.ShapeDtypeStruct((TOKENS, D_HID), jnp.float32),
        compiler_params=pltpu.CompilerParams(
            dimension_semantics=("arbitrary",)),
    )(i0, i1, xbf, W, W, b3, b3, s0, s1)
    return out


# speedup vs baseline: 1.0679x; 1.0009x over previous
import jax
import jax.numpy as jnp
from jax import lax
from jax.experimental import pallas as pl
from jax.experimental.pallas import tpu as pltpu
from jax.experimental.pallas import tpu_sc as plsc

TOKENS = 4096
D_IN = 2048
D_HID = 2048
N_EXP = 8
K_TOP = 2

BM_A = 1024
BN_C = 256
BM_C = 2048

SC_NC = 2
SC_NS = 16
SC_L = 16
SC_NW = SC_NC * SC_NS
SC_TPW = TOKENS // SC_NW


def _gating_body(x_ref, wg_ref, bg_ref, pt_ref, xbf_ref):
    xv = x_ref[...]
    logits = jnp.dot(xv, wg_ref[...], preferred_element_type=jnp.float32)
    logits = logits + bg_ref[...]
    m = jnp.max(logits, axis=1, keepdims=True)
    e = jnp.exp(logits - m)
    p = e / jnp.sum(e, axis=1, keepdims=True)
    pt_ref[...] = p.T
    xbf_ref[...] = xv.astype(jnp.bfloat16)


def _router_body(pt_hbm, s0_hbm, s1_hbm, i0_hbm, i1_hbm, pv, sv0, sv1,
                 iv0, iv1):
    wid = lax.axis_index("s") * SC_NC + lax.axis_index("c")
    base = wid * SC_TPW
    pltpu.sync_copy(pt_hbm.at[:, pl.ds(base, SC_TPW)], pv)
    lane = lax.iota(jnp.int32, SC_L)
    neg_inf = jnp.float32(-jnp.inf)
    for j in range(SC_TPW // SC_L):
        vs = [pv[e, pl.ds(j * SC_L, SC_L)] for e in range(N_EXP)]
        m1 = vs[0]
        for e in range(1, N_EXP):
            m1 = jnp.maximum(m1, vs[e])
        fi = jnp.full((SC_L,), N_EXP, jnp.int32)
        for e in range(N_EXP):
            fi = jnp.minimum(fi, jnp.where(vs[e] == m1, e, N_EXP))
        m2 = jnp.full((SC_L,), neg_inf)
        for e in range(N_EXP):
            m2 = jnp.maximum(m2, jnp.where(fi == e, neg_inf, vs[e]))
        sv0[pl.ds(j * SC_L, SC_L)] = m1
        sv1[pl.ds(j * SC_L, SC_L)] = m2
        if j == 0:
            @pl.when(wid == 0)
            def _():
                si = jnp.full((SC_L,), N_EXP, jnp.int32)
                for e in range(N_EXP):
                    si = jnp.minimum(
                        si, jnp.where((vs[e] == m2) & (fi != e), e, N_EXP))
                iv0[...] = fi
                iv1[...] = si
                pltpu.sync_copy(iv0, i0_hbm)
                pltpu.sync_copy(iv1, i1_hbm)
    pltpu.sync_copy(sv0, s0_hbm.at[pl.ds(base, SC_TPW)])
    pltpu.sync_copy(sv1, s1_hbm.at[pl.ds(base, SC_TPW)])


def _expert_body(i0_ref, i1_ref, x_ref, w0_ref, w1_ref, b0_ref, b1_ref,
                 s0_ref, s1_ref, o_ref):
    xb = x_ref[...]
    d0 = jnp.dot(xb, w0_ref[0].astype(jnp.bfloat16),
                 preferred_element_type=jnp.float32)
    d1 = jnp.dot(xb, w1_ref[0].astype(jnp.bfloat16),
                 preferred_element_type=jnp.float32)
    s0 = s0_ref[...][:, None]
    s1 = s1_ref[...][:, None]
    o_ref[...] = s0 * (d0 + b0_ref[0]) + s1 * (d1 + b1_ref[0])


def kernel(x, Wg, bg, W, b):
    bg2 = bg.reshape(1, N_EXP)
    b3 = b.reshape(N_EXP, 1, D_HID)

    n_blk = TOKENS // BM_A
    probs_t, xbf = pl.pallas_call(
        _gating_body,
        grid=(n_blk,),
        in_specs=[
            pl.BlockSpec((BM_A, D_IN), lambda i: (i, 0)),
            pl.BlockSpec((D_IN, N_EXP), lambda i: (0, 0)),
            pl.BlockSpec((1, N_EXP), lambda i: (0, 0)),
        ],
        out_specs=[
            pl.BlockSpec((N_EXP, BM_A), lambda i: (0, i)),
            pl.BlockSpec((BM_A, D_IN), lambda i: (i, 0)),
        ],
        out_shape=[
            jax.ShapeDtypeStruct((N_EXP, TOKENS), jnp.float32),
            jax.ShapeDtypeStruct((TOKENS, D_IN), jnp.bfloat16),
        ],
        compiler_params=pltpu.CompilerParams(
            dimension_semantics=("arbitrary",)),
    )(x, Wg, bg2)

    s0, s1, i0, i1 = pl.kernel(
        _router_body,
        out_type=[
            jax.ShapeDtypeStruct((TOKENS,), jnp.float32),
            jax.ShapeDtypeStruct((TOKENS,), jnp.float32),
            jax.ShapeDtypeStruct((SC_L,), jnp.int32),
            jax.ShapeDtypeStruct((SC_L,), jnp.int32),
        ],
        mesh=plsc.VectorSubcoreMesh(core_axis_name="c", subcore_axis_name="s"),
        scratch_types=[
            pltpu.VMEM((N_EXP, SC_TPW), jnp.float32),
            pltpu.VMEM((SC_TPW,), jnp.float32),
            pltpu.VMEM((SC_TPW,), jnp.float32),
            pltpu.VMEM((SC_L,), jnp.int32),
            pltpu.VMEM((SC_L,), jnp.int32),
        ],
    )(probs_t)

    out = pl.pallas_call(
        _expert_body,
        grid_spec=pltpu.PrefetchScalarGridSpec(
            num_scalar_prefetch=2,
            grid=(TOKENS // BM_C, D_HID // BN_C),
            in_specs=[
                pl.BlockSpec((BM_C, D_IN), lambda m, n, i0, i1: (m, 0)),
                pl.BlockSpec((1, D_IN, BN_C),
                             lambda m, n, i0, i1: (i0[0], 0, n)),
                pl.BlockSpec((1, D_IN, BN_C),
                             lambda m, n, i0, i1: (i1[0], 0, n)),
                pl.BlockSpec((1, 1, BN_C), lambda m, n, i0, i1: (i0[0], 0, n)),
                pl.BlockSpec((1, 1, BN_C), lambda m, n, i0, i1: (i1[0], 0, n)),
                pl.BlockSpec((BM_C,), lambda m, n, i0, i1: (m,)),
                pl.BlockSpec((BM_C,), lambda m, n, i0, i1: (m,)),
            ],
            out_specs=pl.BlockSpec((BM_C, BN_C), lambda m, n, i0, i1: (m, n)),
        ),
        out_shape=jax.ShapeDtypeStruct((TOKENS, D_HID), jnp.float32),
        compiler_params=pltpu.CompilerParams(
            dimension_semantics=("arbitrary", "arbitrary")),
    )(i0, i1, xbf, W, W, b3, b3, s0, s1)
    return out
